# Initial kernel scaffold; baseline (speedup 1.0000x reference)
import jax
import jax.numpy as jnp
from jax.experimental import pallas as pl


def kernel(x, W1, b1, W2, b2, edge_index):
    raise NotImplementedError("write your pallas kernel here")


# speedup vs baseline: 11.7522x; 11.7522x over previous
import functools

import jax
import jax.numpy as jnp
from jax import lax
from jax.experimental import pallas as pl
from jax.experimental.pallas import tpu as pltpu
from jax.experimental.pallas import tpu_sc as plsc

N = 100000
E = 1600000
K = 128
ROWS = 12544
E_PAD = ROWS * K
N_SH = 100352
PAD_SPREAD = 352
TPB = ROWS // 32
HID = 32
HALF = 16
BN = 2000

_mesh = plsc.VectorSubcoreMesh(core_axis_name="c", subcore_axis_name="s")
_SC_PARAMS = pltpu.CompilerParams(use_tc_tiling_on_sc=False)

SH_PER = N_SH // 16


@functools.partial(
    pl.kernel,
    out_type=[jax.ShapeDtypeStruct((N_SH,), jnp.float32),
              jax.ShapeDtypeStruct((N_SH,), jnp.float32)],
    mesh=_mesh,
    scratch_types=[
        pltpu.VMEM((TPB, K), jnp.int32),
        pltpu.VMEM((K,), jnp.float32),
        pltpu.VMEM((SH_PER,), jnp.float32),
        pltpu.SemaphoreType.DMA,
        pltpu.VMEM_SHARED((N_SH,), jnp.float32),
    ],
    compiler_params=_SC_PARAMS,
)
def _deg_kernel(dst_hbm, deg_out0, deg_out1, dstbuf, ones_v, zbuf, sem, deg_sh):
    c = lax.axis_index("c")
    s = lax.axis_index("s")

    def _z(i, _):
        zbuf[pl.ds(i * 16, 16)] = jnp.zeros((16,), jnp.float32)
        return 0
    lax.fori_loop(0, SH_PER // 16, _z, 0)
    for k in range(K // 16):
        ones_v[pl.ds(k * 16, 16)] = jnp.ones((16,), jnp.float32)
    pltpu.sync_copy(zbuf, deg_sh.at[pl.ds(s * SH_PER, SH_PER)])
    plsc.subcore_barrier()

    row0 = (c * 16 + s) * TPB
    pltpu.sync_copy(dst_hbm.at[pl.ds(row0, TPB)], dstbuf)

    def _body(i, _):
        pltpu.async_copy(ones_v, deg_sh.at[dstbuf.at[i]], sem, add=True)

        @pl.when(i > 0)
        def _():
            pltpu.make_async_copy(ones_v, deg_sh.at[dstbuf.at[0]], sem).wait()
        return 0
    lax.fori_loop(0, TPB, _body, 0)
    pltpu.make_async_copy(ones_v, deg_sh.at[dstbuf.at[0]], sem).wait()
    plsc.subcore_barrier()

    for cc, out in ((0, deg_out0), (1, deg_out1)):
        @pl.when(c == cc)
        def _():
            pltpu.sync_copy(deg_sh.at[pl.ds(s * SH_PER, SH_PER)],
                            out.at[pl.ds(s * SH_PER, SH_PER)])


STAGES = (200, 200, 200, 184)
SBUF = STAGES[0]
NHALF = 50000
ACC_R = 51200
ACC_PER = ACC_R // 16
DUM0 = 50176


@functools.partial(
    pl.kernel,
    out_type=jax.ShapeDtypeStruct((2, 2, ACC_R, HALF), jnp.float32),
    mesh=_mesh,
    scratch_types=[
        pltpu.VMEM((SBUF, K), jnp.int32),
        pltpu.VMEM((SBUF, K), jnp.int32),
        pltpu.VMEM((2, K), jnp.int32),
        pltpu.VMEM((K,), jnp.int32),
        pltpu.VMEM((K, HALF), jnp.float32),
        pltpu.VMEM((K, HALF), jnp.float32),
        pltpu.VMEM((ACC_PER // 4, HALF), jnp.float32),
        pltpu.SemaphoreType.DMA,
        pltpu.SemaphoreType.DMA,
        pltpu.SemaphoreType.DMA,
        pltpu.SemaphoreType.DMA,
        pltpu.VMEM_SHARED((ACC_R, HALF), jnp.float32),
    ],
    compiler_params=_SC_PARAMS,
)
def _prop_kernel(g_hbm, src_hbm, dst_hbm, acc_out,
                 srcbuf, dstbuf, lbuf, rbuf, rows0, rows1, zbuf,
                 gs0, gs1, ss0, ss1, acc_sh):
    c = lax.axis_index("c")
    s = lax.axis_index("s")
    rows = (rows0, rows1)
    gs = (gs0, gs1)
    ss = (ss0, ss1)
    gsrc = g_hbm.at[c]

    for q in range(K // 16):
        rbuf[pl.ds(q * 16, 16)] = lax.iota(jnp.int32, 16) + q * 16

    def _z(i, _):
        zbuf[i, :] = jnp.zeros((16,), jnp.float32)
        return 0
    lax.fori_loop(0, ACC_PER // 4, _z, 0)

    for h in (0, 1):
        for k in range(4):
            pltpu.sync_copy(
                zbuf,
                acc_sh.at[pl.ds(s * ACC_PER + k * (ACC_PER // 4),
                                ACC_PER // 4)])
        plsc.subcore_barrier()

        def _lidx(i, bslot):
            for q in range(K // 16):
                d = dstbuf[i, pl.ds(q * 16, 16)]
                l = d - (h * NHALF)
                dum = DUM0 + ((rbuf[pl.ds(q * 16, 16)] + i * 16) & 1023)
                oob = (l < 0) | (l >= NHALF)
                lbuf[bslot, pl.ds(q * 16, 16)] = jnp.where(oob, dum, l)

        st_off = 0
        for nr in STAGES:
            r0 = s * (ROWS // 16) + st_off
            st_off += nr
            pltpu.sync_copy(src_hbm.at[pl.ds(r0, nr)], srcbuf.at[pl.ds(0, nr)])
            pltpu.sync_copy(dst_hbm.at[pl.ds(r0, nr)], dstbuf.at[pl.ds(0, nr)])

            _lidx(0, 0)
            pltpu.async_copy(gsrc.at[srcbuf.at[0]], rows0, gs0)

            def _pair(ii, _):
                for b in (0, 1):
                    i = 2 * ii + b
                    pltpu.make_async_copy(gsrc.at[srcbuf.at[0]], rows[b],
                                          gs[b]).wait()

                    @pl.when(i > 0)
                    def _():
                        pltpu.make_async_copy(
                            rows[1 - b], acc_sh.at[lbuf.at[1 - b]],
                            ss[1 - b]).wait()

                    @pl.when(i + 1 < nr)
                    def _():
                        _lidx(i + 1, 1 - b)
                        pltpu.async_copy(gsrc.at[srcbuf.at[i + 1]],
                                         rows[1 - b], gs[1 - b])
                    pltpu.async_copy(rows[b], acc_sh.at[lbuf.at[b]], ss[b],
                                     add=True)
                return 0
            lax.fori_loop(0, nr // 2, _pair, 0)
            pltpu.make_async_copy(rows1, acc_sh.at[lbuf.at[1]], ss1).wait()

        plsc.subcore_barrier()
        pltpu.sync_copy(acc_sh.at[pl.ds(s * ACC_PER, ACC_PER)],
                        acc_out.at[c, h, pl.ds(s * ACC_PER, ACC_PER)])
        plsc.subcore_barrier()


def _tc1_body(x_ref, w_ref, d0_ref, d1_ref, g_ref):
    h = jnp.dot(x_ref[...], w_ref[...], preferred_element_type=jnp.float32)
    sc = lax.rsqrt(d0_ref[...] + d1_ref[...] + 1.0)
    g = h * sc
    g_ref[0] = g[:, :HALF]
    g_ref[1] = g[:, HALF:]


def _tc2_body(acc_ref, g_ref, d0_ref, d1_ref, w_ref, b_ref, out_ref):
    sc = lax.rsqrt(d0_ref[...] + d1_ref[...] + 1.0)
    a = jnp.concatenate([acc_ref[0], acc_ref[1]], axis=1)
    g = jnp.concatenate([g_ref[0], g_ref[1]], axis=1)
    o1 = jnp.maximum(sc * (a + g) + b_ref[...], 0.0)
    h2 = jnp.dot(o1, w_ref[...], preferred_element_type=jnp.float32)
    g2 = h2 * sc
    out_ref[0] = g2[:, :HALF]
    out_ref[1] = g2[:, HALF:]


def _tc3_body(acc_ref, g_ref, d0_ref, d1_ref, b_ref, out_ref):
    sc = lax.rsqrt(d0_ref[...] + d1_ref[...] + 1.0)
    a = jnp.concatenate([acc_ref[0], acc_ref[1]], axis=1)
    g = jnp.concatenate([g_ref[0], g_ref[1]], axis=1)
    out_ref[...] = sc * (a + g) + b_ref[...]


_SPLIT = pl.BlockSpec((2, BN, HALF), lambda i: (0, i, 0))
_DP = pl.BlockSpec((BN, 1), lambda i: (i, 0))
_BIAS = pl.BlockSpec((1, HID), lambda i: (0, 0))
_GRID = N // BN


def _tc1(x, W1, d0, d1):
    return pl.pallas_call(
        _tc1_body,
        grid=(_GRID,),
        in_specs=[pl.BlockSpec((BN, 128), lambda i: (i, 0)),
                  pl.BlockSpec((128, HID), lambda i: (0, 0)),
                  _DP, _DP],
        out_specs=_SPLIT,
        out_shape=jax.ShapeDtypeStruct((2, N, HALF), jnp.float32),
    )(x, W1, d0, d1)


def _tc2(acc1, g1, d0, d1, W2, b1):
    return pl.pallas_call(
        _tc2_body,
        grid=(_GRID,),
        in_specs=[_SPLIT, _SPLIT, _DP, _DP,
                  pl.BlockSpec((HID, HID), lambda i: (0, 0)),
                  _BIAS],
        out_specs=_SPLIT,
        out_shape=jax.ShapeDtypeStruct((2, N, HALF), jnp.float32),
    )(acc1, g1, d0, d1, W2, b1)


def _tc3(acc2, g2, d0, d1, b2):
    return pl.pallas_call(
        _tc3_body,
        grid=(_GRID,),
        in_specs=[_SPLIT, _SPLIT, _DP, _DP, _BIAS],
        out_specs=pl.BlockSpec((BN, HID), lambda i: (i, 0)),
        out_shape=jax.ShapeDtypeStruct((N, HID), jnp.float32),
    )(acc2, g2, d0, d1, b2)


def kernel(x, W1, b1, W2, b2, edge_index):
    ei = edge_index.astype(jnp.int32)
    npad = E_PAD - E
    spread = jnp.arange(npad, dtype=jnp.int32) % PAD_SPREAD
    src2d = jnp.concatenate([ei[0], spread]).reshape(ROWS, K)
    dst2d = jnp.concatenate([ei[1], N + spread]).reshape(ROWS, K)

    dp0, dp1 = _deg_kernel(dst2d)
    d0 = dp0[:N].reshape(N, 1)
    d1 = dp1[:N].reshape(N, 1)

    def _prop(g):
        a4 = _prop_kernel(g, src2d, dst2d)
        return jnp.concatenate([a4[:, 0, :NHALF], a4[:, 1, :NHALF]], axis=1)

    g1 = _tc1(x, W1, d0, d1)
    acc1 = _prop(g1)
    g2 = _tc2(acc1, g1, d0, d1, W2, b1.reshape(1, HID))
    acc2 = _prop(g2)
    return _tc3(acc2, g2, d0, d1, b2.reshape(1, HID))
